# baseline (speedup 1.0000x reference)
import functools

import jax
import jax.numpy as jnp
from jax import lax
from jax.experimental import pallas as pl
from jax.experimental.pallas import tpu as pltpu
from jax.experimental.pallas import tpu_sc as plsc

_LANES = 16
_G = 128


@functools.lru_cache(maxsize=None)
def _make_sc_embed(batch: int):
    info = plsc.get_sparse_core_info()
    nc, ns = info.num_cores, info.num_subcores
    nw = nc * ns
    assert batch % (nw * _G) == 0
    groups_per_w = batch // (nw * _G)
    x_per_w = groups_per_w * 2 * _G
    o_per_w = groups_per_w * 4 * _G

    mesh = plsc.VectorSubcoreMesh(core_axis_name="c", subcore_axis_name="s")

    @functools.partial(
        pl.kernel,
        out_type=jax.ShapeDtypeStruct((batch // _G * 4 * _G,), jnp.float32),
        mesh=mesh,
        scratch_types=[
            pltpu.VMEM((x_per_w,), jnp.int32),
            pltpu.VMEM((2, 4), jnp.float32),
            pltpu.VMEM((2,), jnp.float32),
            pltpu.VMEM((o_per_w,), jnp.float32),
            pltpu.SemaphoreType.DMA,
            pltpu.SemaphoreType.DMA,
            pltpu.SemaphoreType.DMA,
        ],
        compiler_params=pltpu.CompilerParams(
            needs_layout_passes=False, skip_device_barrier=True
        ),
    )
    def sc_embed(x_hbm, ws_hbm, wy_hbm, out_hbm, x_v, ws_v, wy_v, out_v,
                 sem_x, sem_s, sem_y):
        wid = lax.axis_index("s") * nc + lax.axis_index("c")
        cp_x = pltpu.async_copy(
            x_hbm.at[pl.ds(wid * x_per_w, x_per_w)], x_v, sem_x)
        cp_s = pltpu.async_copy(ws_hbm, ws_v, sem_s)
        cp_y = pltpu.async_copy(wy_hbm, wy_v, sem_y)
        cp_s.wait()
        cp_y.wait()
        cp_x.wait()
        col0 = jnp.zeros((_LANES,), jnp.int32)
        col1 = col0 + 1
        chunks = _G // _LANES

        @pl.loop(0, groups_per_w * chunks)
        def _(i):
            gl = i // chunks
            r = (i % chunks) * _LANES
            xo = gl * 2 * _G + r
            x0 = x_v[pl.ds(xo, _LANES)]
            x1 = x_v[pl.ds(xo + _G, _LANES)]
            c0 = plsc.load_gather(ws_v, [col0, x0])
            c1 = plsc.load_gather(ws_v, [col1, x0])
            c2 = plsc.load_gather(wy_v, [x1])
            o = gl * 4 * _G + r
            out_v[pl.ds(o, _LANES)] = c0
            out_v[pl.ds(o + _G, _LANES)] = c1
            out_v[pl.ds(o + 2 * _G, _LANES)] = c2
        pltpu.sync_copy(out_v, out_hbm.at[pl.ds(wid * o_per_w, o_per_w)])

    return sc_embed


def kernel(x, W_y, W_s):
    batch = x.shape[0]
    ng = batch // _G
    x_flat = (
        x.astype(jnp.int32).reshape(ng, _G, 2).swapaxes(1, 2).reshape(-1)
    )
    ws_t = W_s.astype(jnp.float32).T
    wy_flat = W_y.astype(jnp.float32).reshape(-1)
    out_flat = _make_sc_embed(batch)(x_flat, ws_t, wy_flat)
    out4 = out_flat.reshape(ng, 4, _G).swapaxes(1, 2).reshape(batch, 4)
    return out4[:, :3]


# speedup vs baseline: 1.0092x; 1.0092x over previous
import functools

import jax
import jax.numpy as jnp
from jax import lax
from jax.experimental import pallas as pl
from jax.experimental.pallas import tpu as pltpu
from jax.experimental.pallas import tpu_sc as plsc

_LANES = 16
_G = 128


@functools.lru_cache(maxsize=None)
def _make_sc_embed(batch: int):
    info = plsc.get_sparse_core_info()
    nc, ns = info.num_cores, info.num_subcores
    nw = nc * ns
    assert batch % (nw * _G) == 0
    groups_per_w = batch // (nw * _G)
    x_per_w = groups_per_w * 2 * _G
    o_per_w = groups_per_w * 4 * _G

    mesh = plsc.VectorSubcoreMesh(core_axis_name="c", subcore_axis_name="s")

    @functools.partial(
        pl.kernel,
        out_type=jax.ShapeDtypeStruct((batch // _G * 4 * _G,), jnp.float32),
        mesh=mesh,
        scratch_types=[
            pltpu.VMEM((x_per_w,), jnp.int32),
            pltpu.VMEM((2, 4), jnp.float32),
            pltpu.VMEM((2,), jnp.float32),
            pltpu.VMEM((o_per_w,), jnp.float32),
            pltpu.SemaphoreType.DMA,
            pltpu.SemaphoreType.DMA,
            pltpu.SemaphoreType.DMA,
        ],
        compiler_params=pltpu.CompilerParams(needs_layout_passes=False),
    )
    def sc_embed(x_hbm, ws_hbm, wy_hbm, out_hbm, x_v, ws_v, wy_v, out_v,
                 sem_x, sem_s, sem_y):
        wid = lax.axis_index("s") * nc + lax.axis_index("c")
        cp_x = pltpu.async_copy(
            x_hbm.at[pl.ds(wid * x_per_w, x_per_w)], x_v, sem_x)
        cp_s = pltpu.async_copy(ws_hbm, ws_v, sem_s)
        cp_y = pltpu.async_copy(wy_hbm, wy_v, sem_y)
        cp_s.wait()
        cp_y.wait()
        cp_x.wait()
        col0 = jnp.zeros((_LANES,), jnp.int32)
        col1 = col0 + 1
        chunks = _G // _LANES

        @pl.loop(0, groups_per_w * chunks, unroll=2)
        def _(i):
            gl = i // chunks
            r = (i % chunks) * _LANES
            xo = gl * 2 * _G + r
            x0 = x_v[pl.ds(xo, _LANES)]
            x1 = x_v[pl.ds(xo + _G, _LANES)]
            c0 = plsc.load_gather(ws_v, [col0, x0])
            c1 = plsc.load_gather(ws_v, [col1, x0])
            c2 = plsc.load_gather(wy_v, [x1])
            o = gl * 4 * _G + r
            out_v[pl.ds(o, _LANES)] = c0
            out_v[pl.ds(o + _G, _LANES)] = c1
            out_v[pl.ds(o + 2 * _G, _LANES)] = c2
        pltpu.sync_copy(out_v, out_hbm.at[pl.ds(wid * o_per_w, o_per_w)])

    return sc_embed


def kernel(x, W_y, W_s):
    batch = x.shape[0]
    ng = batch // _G
    x_flat = (
        x.astype(jnp.int32).reshape(ng, _G, 2).swapaxes(1, 2).reshape(-1)
    )
    ws_t = W_s.astype(jnp.float32).T
    wy_flat = W_y.astype(jnp.float32).reshape(-1)
    out_flat = _make_sc_embed(batch)(x_flat, ws_t, wy_flat)
    out4 = out_flat.reshape(ng, 4, _G).swapaxes(1, 2).reshape(batch, 4)
    return out4[:, :3]
